# baseline (speedup 1.0000x reference)
import functools
import math

import jax
import jax.numpy as jnp
from jax import lax
from jax.experimental import pallas as pl
from jax.experimental.pallas import tpu as pltpu
from jax.experimental.pallas import tpu_sc as plsc

D_MODEL = 64
SCALE = math.sqrt(D_MODEL)

NUM_CORES = 2
NUM_SUBCORES = 16
NUM_WORKERS = NUM_CORES * NUM_SUBCORES

IDX_MINOR = 128
CHUNK = 512
K = CHUNK // IDX_MINOR


def _emb_body(x_hbm, lut_hbm, out_hbm, idx_v, rows_v, sem, *, rows_per_worker):
    wid = lax.axis_index("s") * NUM_CORES + lax.axis_index("c")
    base = wid * rows_per_worker
    num_chunks = rows_per_worker // CHUNK

    def chunk_body(i, carry):
        start = base + i * CHUNK
        pltpu.sync_copy(x_hbm.at[pl.ds(start, CHUNK)], idx_v)
        for j in range(K):
            pltpu.async_copy(
                lut_hbm.at[idx_v.at[pl.ds(j * IDX_MINOR, IDX_MINOR)]],
                rows_v.at[pl.ds(j * IDX_MINOR, IDX_MINOR)],
                sem,
            )
        for j in range(K):
            pltpu.make_async_copy(
                lut_hbm.at[idx_v.at[pl.ds(j * IDX_MINOR, IDX_MINOR)]],
                rows_v.at[pl.ds(j * IDX_MINOR, IDX_MINOR)],
                sem,
            ).wait()

        def scale_row(r, c):
            for v in range(D_MODEL // 16):
                sl = pl.ds(v * 16, 16)
                rows_v[r, sl] = rows_v[r, sl] * SCALE
            return c

        lax.fori_loop(0, CHUNK, scale_row, 0, unroll=2)

        pltpu.sync_copy(rows_v, out_hbm.at[pl.ds(start, CHUNK)])
        return carry

    lax.fori_loop(0, num_chunks, chunk_body, 0)


def kernel(x, lut):
    b, s = x.shape
    n = b * s
    assert n % (NUM_WORKERS * CHUNK) == 0
    rows_per_worker = n // NUM_WORKERS
    x_flat = x.reshape(n)

    mesh = plsc.VectorSubcoreMesh(core_axis_name="c", subcore_axis_name="s")
    run = pl.kernel(
        functools.partial(_emb_body, rows_per_worker=rows_per_worker),
        out_type=jax.ShapeDtypeStruct((n, D_MODEL), jnp.float32),
        mesh=mesh,
        scratch_types=[
            pltpu.VMEM((CHUNK,), jnp.int32),
            pltpu.VMEM((CHUNK, D_MODEL), jnp.float32),
            pltpu.SemaphoreType.DMA,
        ],
        compiler_params=pltpu.CompilerParams(use_tc_tiling_on_sc=False),
    )
    out = run(x_flat, lut)
    return out.reshape(b, s, D_MODEL)
